# baseline (speedup 1.0000x reference)
import math

import jax
import jax.numpy as jnp
from jax import lax
from jax.experimental import pallas as pl

B, N, D, H = 16, 256, 256, 8
HD = D // H
S = 8
SH = S * H
KC = 6
PROX = 20.0
SCALE = math.sqrt(float(HD))
SPB = 8

_DN = (((1,), (1,)), ((), ()))
_DNS = (((1,), (0,)), ((), ()))


def _dot(a, b, dn=_DNS):
    return lax.dot_general(a, b, dn, preferred_element_type=jnp.float32)


def _fused_kernel(dist_full_ref, speed_ref, dist_col_ref, mask_col_ref,
                  tokens_ref, qw2_ref, kw_ref, vw_ref, ow_ref,
                  w1t_ref, b1r_ref, w2_ref, b2r_ref, out_ref):
    pid = pl.program_id(0)

    dist_all = dist_full_ref[...]
    close = jnp.sum((dist_all < PROX).astype(jnp.float32))
    avg_density = close / (B * N)
    avg_speed = jnp.mean(speed_ref[...])
    K_t = (4
           + (avg_speed > 15.0).astype(jnp.int32)
           + (avg_density > 0.5).astype(jnp.int32))
    K_t = jnp.minimum(K_t, KC)

    tok_flat = tokens_ref[...].reshape(SPB * N, D)
    q2_all = _dot(tok_flat, qw2_ref[...], _DN)
    mask_flat = mask_col_ref[...].reshape(SPB * N, 1)
    q_all = jnp.where(mask_flat > 0.0, q2_all[:, D:], q2_all[:, :D])
    dist_flat = dist_col_ref[...].reshape(SPB * N, 1)

    lane_j_col = lax.broadcasted_iota(jnp.int32, (D, SH), 1)
    d_iota_col = lax.broadcasted_iota(jnp.int32, (D, SH), 0)
    hm = ((d_iota_col // HD) == (lane_j_col % H)).astype(jnp.float32)
    rep = (lax.broadcasted_iota(jnp.int32, (S, SH), 1) // H
           == lax.broadcasted_iota(jnp.int32, (S, SH), 0)
           ).astype(jnp.float32)
    lane_j_row = lax.broadcasted_iota(jnp.int32, (SH, D), 0)
    d_iota_row = lax.broadcasted_iota(jnp.int32, (SH, D), 1)
    hmt = ((d_iota_row // HD) == (lane_j_row % H)).astype(jnp.float32)
    rept = ((lax.broadcasted_iota(jnp.int32, (SH, S), 0) // H)
            == lax.broadcasted_iota(jnp.int32, (SH, S), 1)
            ).astype(jnp.float32)
    g = ((lax.broadcasted_iota(jnp.int32, (SH, SH), 0) % H)
         == (lax.broadcasted_iota(jnp.int32, (SH, SH), 1) % H)
         ).astype(jnp.float32)
    sel_rows = (lax.broadcasted_iota(jnp.int32, ((KC + 1) * N, S), 0) // N
                == lax.broadcasted_iota(jnp.int32, ((KC + 1) * N, S), 1)
                ).astype(jnp.float32)
    sub_nn = lax.broadcasted_iota(jnp.int32, (N, N), 0)
    lan_nn = lax.broadcasted_iota(jnp.int32, (N, N), 1)
    row_s = lax.broadcasted_iota(jnp.int32, (S, N), 0)
    lane_sh = lax.broadcasted_iota(jnp.int32, (N, SH), 1)
    svals = lane_sh // H

    attn_scenes = []
    for sc in range(SPB):
        bg = pid * SPB + sc
        d_row = dist_full_ref[pl.ds(bg, 1), :]
        d_col = dist_flat[sc * N:(sc + 1) * N, :]
        tokens = tok_flat[sc * N:(sc + 1) * N, :]
        q = q_all[sc * N:(sc + 1) * N, :]

        lt = d_row < d_col
        eq = d_row == d_col
        cmp_t = (lt | (eq & (lan_nn < sub_nn))).astype(jnp.float32)
        rank_col = jnp.sum(cmp_t, axis=1, keepdims=True).astype(jnp.int32)
        lt2 = d_col < d_row
        cmp_r = (lt2 | (eq & (sub_nn < lan_nn))).astype(jnp.float32)
        rank_row = jnp.sum(cmp_r, axis=0, keepdims=True).astype(jnp.int32)

        r_col = jnp.minimum(rank_col, S - 1)
        onehot = ((row_s == rank_row) & (rank_row < KC + 1)
                  ).astype(jnp.float32)

        cand_tok = _dot(onehot, tokens)
        d_cand = _dot(onehot, d_col)

        kt = _dot(kw_ref[...], cand_tok, _DN)
        v_cand = _dot(cand_tok, vw_ref[...], _DN)
        khead = _dot(kt, rep) * hm
        vheadt = _dot(rept, v_cand) * hmt

        sf = _dot(q, khead) * (1.0 / SCALE)

        a_term = d_col * w1t_ref[0:1, :] + b1r_ref[...]
        a_rows = jnp.concatenate([a_term] * (KC + 1), axis=0)
        b_rows = _dot(sel_rows, d_cand)
        h1_rows = jnp.maximum(a_rows + b_rows * w1t_ref[1:2, :], 0.0)
        bias_rows = _dot(h1_rows, w2_ref[...], _DN) + b2r_ref[...]
        bias_all = jnp.concatenate(
            [bias_rows[s * N:(s + 1) * N, :] for s in range(KC + 1)]
            + [jnp.zeros((N, H), jnp.float32)], axis=1)

        rank_after_self = svals - (svals > r_col).astype(jnp.int32)
        valid = ((svals != r_col) & (rank_after_self < K_t)
                 & (svals < KC + 1))
        validf = valid.astype(jnp.float32)

        z = jnp.where(valid, sf + bias_all, -1e30)
        m = jnp.max(z, axis=1, keepdims=True)
        e = jnp.exp(z - m) * validf
        denom = _dot(e, g)
        w_all = e / denom

        attn_scenes.append(_dot(w_all, vheadt))

    attn_all = jnp.concatenate(attn_scenes, axis=0)
    out_ref[...] = _dot(attn_all, ow_ref[...], _DN).reshape(SPB, N, D)


@jax.jit
def kernel(tokens_B, ego_distances, ego_mask, ego_speed, q_w, k_w, v_w,
           ego_q_w, ego_k_w, ego_v_w, out_w, w1, b1, w2, b2):
    del ego_k_w, ego_v_w
    speed_row = ego_speed.reshape(1, B)
    dist_col = ego_distances.reshape(B, N, 1)
    mask_col = ego_mask.astype(jnp.float32).reshape(B, N, 1)
    qw2 = jnp.concatenate([q_w, ego_q_w], axis=0)
    w1t = w1.T
    b1r = b1.reshape(1, -1)
    b2r = b2.reshape(1, -1)

    const = lambda b: (0, 0)
    grid_spec = pl.GridSpec(
        grid=(B // SPB,),
        in_specs=[
            pl.BlockSpec((B, N), const),
            pl.BlockSpec((1, B), const),
            pl.BlockSpec((SPB, N, 1), lambda b: (b, 0, 0)),
            pl.BlockSpec((SPB, N, 1), lambda b: (b, 0, 0)),
            pl.BlockSpec((SPB, N, D), lambda b: (b, 0, 0)),
            pl.BlockSpec((2 * D, D), const),
            pl.BlockSpec((D, D), const),
            pl.BlockSpec((D, D), const),
            pl.BlockSpec((D, D), const),
            pl.BlockSpec((2, D // 4), const),
            pl.BlockSpec((1, D // 4), const),
            pl.BlockSpec((H, D // 4), const),
            pl.BlockSpec((1, H), const),
        ],
        out_specs=pl.BlockSpec((SPB, N, D), lambda b: (b, 0, 0)),
    )
    return pl.pallas_call(
        _fused_kernel,
        grid_spec=grid_spec,
        out_shape=jax.ShapeDtypeStruct((B, N, D), jnp.float32),
    )(ego_distances, speed_row, dist_col, mask_col, tokens_B,
      qw2, k_w, v_w, out_w, w1t, b1r, w2, b2r)


# speedup vs baseline: 1.4044x; 1.4044x over previous
import math

import jax
import jax.numpy as jnp
from jax import lax
from jax.experimental import pallas as pl

B, N, D, H = 16, 256, 256, 8
HD = D // H
S = 8
SH = S * H
KC = 6
PROX = 20.0
SCALE = math.sqrt(float(HD))
SPB = 2

_DN = (((1,), (1,)), ((), ()))
_DNS = (((1,), (0,)), ((), ()))


def _dot(a, b, dn=_DNS):
    return lax.dot_general(a, b, dn, preferred_element_type=jnp.float32)


def _fused_kernel(dist_full_ref, speed_ref, dist_col_ref, mask_col_ref,
                  tokens_ref, qw2_ref, kw_ref, vw_ref, ow_ref,
                  w1t_ref, b1r_ref, w2_ref, b2r_ref, out_ref):
    pid = pl.program_id(0)

    dist_all = dist_full_ref[...]
    close = jnp.sum((dist_all < PROX).astype(jnp.float32))
    avg_density = close / (B * N)
    avg_speed = jnp.mean(speed_ref[...])
    K_t = (4
           + (avg_speed > 15.0).astype(jnp.int32)
           + (avg_density > 0.5).astype(jnp.int32))
    K_t = jnp.minimum(K_t, KC)

    tok_flat = tokens_ref[...].reshape(SPB * N, D)
    q2_all = _dot(tok_flat, qw2_ref[...], _DN)
    mask_flat = mask_col_ref[...].reshape(SPB * N, 1)
    q_all = jnp.where(mask_flat > 0.0, q2_all[:, D:], q2_all[:, :D])
    dist_flat = dist_col_ref[...].reshape(SPB * N, 1)

    lane_j_col = lax.broadcasted_iota(jnp.int32, (D, SH), 1)
    d_iota_col = lax.broadcasted_iota(jnp.int32, (D, SH), 0)
    hm = ((d_iota_col // HD) == (lane_j_col % H)).astype(jnp.float32)
    rep = (lax.broadcasted_iota(jnp.int32, (S, SH), 1) // H
           == lax.broadcasted_iota(jnp.int32, (S, SH), 0)
           ).astype(jnp.float32)
    lane_j_row = lax.broadcasted_iota(jnp.int32, (SH, D), 0)
    d_iota_row = lax.broadcasted_iota(jnp.int32, (SH, D), 1)
    hmt = ((d_iota_row // HD) == (lane_j_row % H)).astype(jnp.float32)
    rept = ((lax.broadcasted_iota(jnp.int32, (SH, S), 0) // H)
            == lax.broadcasted_iota(jnp.int32, (SH, S), 1)
            ).astype(jnp.float32)
    g = ((lax.broadcasted_iota(jnp.int32, (SH, SH), 0) % H)
         == (lax.broadcasted_iota(jnp.int32, (SH, SH), 1) % H)
         ).astype(jnp.float32)
    sel_rows = (lax.broadcasted_iota(jnp.int32, ((KC + 1) * N, S), 0) // N
                == lax.broadcasted_iota(jnp.int32, ((KC + 1) * N, S), 1)
                ).astype(jnp.float32)
    sub_nn = lax.broadcasted_iota(jnp.int32, (N, N), 0)
    lan_nn = lax.broadcasted_iota(jnp.int32, (N, N), 1)
    row_s = lax.broadcasted_iota(jnp.int32, (S, N), 0)
    lane_sh = lax.broadcasted_iota(jnp.int32, (N, SH), 1)
    svals = lane_sh // H

    attn_scenes = []
    for sc in range(SPB):
        bg = pid * SPB + sc
        d_row = dist_full_ref[pl.ds(bg, 1), :]
        d_col = dist_flat[sc * N:(sc + 1) * N, :]
        tokens = tok_flat[sc * N:(sc + 1) * N, :]
        q = q_all[sc * N:(sc + 1) * N, :]

        lt = d_row < d_col
        eq = d_row == d_col
        cmp_t = (lt | (eq & (lan_nn < sub_nn))).astype(jnp.float32)
        rank_col = jnp.sum(cmp_t, axis=1, keepdims=True).astype(jnp.int32)
        lt2 = d_col < d_row
        cmp_r = (lt2 | (eq & (sub_nn < lan_nn))).astype(jnp.float32)
        rank_row = jnp.sum(cmp_r, axis=0, keepdims=True).astype(jnp.int32)

        r_col = jnp.minimum(rank_col, S - 1)
        onehot = ((row_s == rank_row) & (rank_row < KC + 1)
                  ).astype(jnp.float32)

        cand_tok = _dot(onehot, tokens)
        d_cand = _dot(onehot, d_col)

        kt = _dot(kw_ref[...], cand_tok, _DN)
        v_cand = _dot(cand_tok, vw_ref[...], _DN)
        khead = _dot(kt, rep) * hm
        vheadt = _dot(rept, v_cand) * hmt

        sf = _dot(q, khead) * (1.0 / SCALE)

        a_term = d_col * w1t_ref[0:1, :] + b1r_ref[...]
        a_rows = jnp.concatenate([a_term] * (KC + 1), axis=0)
        b_rows = _dot(sel_rows, d_cand)
        h1_rows = jnp.maximum(a_rows + b_rows * w1t_ref[1:2, :], 0.0)
        bias_rows = _dot(h1_rows, w2_ref[...], _DN) + b2r_ref[...]
        bias_all = jnp.concatenate(
            [bias_rows[s * N:(s + 1) * N, :] for s in range(KC + 1)]
            + [jnp.zeros((N, H), jnp.float32)], axis=1)

        rank_after_self = svals - (svals > r_col).astype(jnp.int32)
        valid = ((svals != r_col) & (rank_after_self < K_t)
                 & (svals < KC + 1))
        validf = valid.astype(jnp.float32)

        z = jnp.where(valid, sf + bias_all, -1e30)
        m = jnp.max(z, axis=1, keepdims=True)
        e = jnp.exp(z - m) * validf
        denom = _dot(e, g)
        w_all = e / denom

        attn_scenes.append(_dot(w_all, vheadt))

    attn_all = jnp.concatenate(attn_scenes, axis=0)
    out_ref[...] = _dot(attn_all, ow_ref[...], _DN).reshape(SPB, N, D)


@jax.jit
def kernel(tokens_B, ego_distances, ego_mask, ego_speed, q_w, k_w, v_w,
           ego_q_w, ego_k_w, ego_v_w, out_w, w1, b1, w2, b2):
    del ego_k_w, ego_v_w
    speed_row = ego_speed.reshape(1, B)
    dist_col = ego_distances.reshape(B, N, 1)
    mask_col = ego_mask.astype(jnp.float32).reshape(B, N, 1)
    qw2 = jnp.concatenate([q_w, ego_q_w], axis=0)
    w1t = w1.T
    b1r = b1.reshape(1, -1)
    b2r = b2.reshape(1, -1)

    const = lambda b: (0, 0)
    grid_spec = pl.GridSpec(
        grid=(B // SPB,),
        in_specs=[
            pl.BlockSpec((B, N), const),
            pl.BlockSpec((1, B), const),
            pl.BlockSpec((SPB, N, 1), lambda b: (b, 0, 0)),
            pl.BlockSpec((SPB, N, 1), lambda b: (b, 0, 0)),
            pl.BlockSpec((SPB, N, D), lambda b: (b, 0, 0)),
            pl.BlockSpec((2 * D, D), const),
            pl.BlockSpec((D, D), const),
            pl.BlockSpec((D, D), const),
            pl.BlockSpec((D, D), const),
            pl.BlockSpec((2, D // 4), const),
            pl.BlockSpec((1, D // 4), const),
            pl.BlockSpec((H, D // 4), const),
            pl.BlockSpec((1, H), const),
        ],
        out_specs=pl.BlockSpec((SPB, N, D), lambda b: (b, 0, 0)),
    )
    return pl.pallas_call(
        _fused_kernel,
        grid_spec=grid_spec,
        out_shape=jax.ShapeDtypeStruct((B, N, D), jnp.float32),
    )(ego_distances, speed_row, dist_col, mask_col, tokens_B,
      qw2, k_w, v_w, out_w, w1t, b1r, w2, b2r)


# speedup vs baseline: 1.4871x; 1.0589x over previous
import math

import jax
import jax.numpy as jnp
from jax import lax
from jax.experimental import pallas as pl

B, N, D, H = 16, 256, 256, 8
HD = D // H
S = 8
SH = S * H
KC = 6
PROX = 20.0
SCALE = math.sqrt(float(HD))
SPB = 2

_DN = (((1,), (1,)), ((), ()))
_DNS = (((1,), (0,)), ((), ()))


def _dot(a, b, dn=_DNS):
    return lax.dot_general(a, b, dn, preferred_element_type=jnp.float32)


def _fused_kernel(dist_full_ref, speed_ref, dist_col_ref, mask_col_ref,
                  tokens_ref, qw2_ref, kw_ref, vw_ref, ow_ref,
                  w1t_ref, b1r_ref, w2_ref, b2r_ref, out_ref):
    pid = pl.program_id(0)

    dist_all = dist_full_ref[...]
    close = jnp.sum((dist_all < PROX).astype(jnp.float32))
    avg_density = close / (B * N)
    avg_speed = jnp.mean(speed_ref[...])
    K_t = (4
           + (avg_speed > 15.0).astype(jnp.int32)
           + (avg_density > 0.5).astype(jnp.int32))
    K_t = jnp.minimum(K_t, KC)

    tok_flat = tokens_ref[...].reshape(SPB * N, D)
    q2_all = _dot(tok_flat, qw2_ref[...], _DN)
    mask_flat = mask_col_ref[...].reshape(SPB * N, 1)
    q_all = jnp.where(mask_flat > 0.0, q2_all[:, D:], q2_all[:, :D])
    dist_flat = dist_col_ref[...].reshape(SPB * N, 1)

    lane_j_col = lax.broadcasted_iota(jnp.int32, (D, SH), 1)
    d_iota_col = lax.broadcasted_iota(jnp.int32, (D, SH), 0)
    hm = ((d_iota_col // HD) == (lane_j_col % H)).astype(jnp.float32)
    rep = (lax.broadcasted_iota(jnp.int32, (S, SH), 1) // H
           == lax.broadcasted_iota(jnp.int32, (S, SH), 0)
           ).astype(jnp.float32)
    lane_j_row = lax.broadcasted_iota(jnp.int32, (SH, D), 0)
    d_iota_row = lax.broadcasted_iota(jnp.int32, (SH, D), 1)
    hmt = ((d_iota_row // HD) == (lane_j_row % H)).astype(jnp.float32)
    rept = ((lax.broadcasted_iota(jnp.int32, (SH, S), 0) // H)
            == lax.broadcasted_iota(jnp.int32, (SH, S), 1)
            ).astype(jnp.float32)
    g = ((lax.broadcasted_iota(jnp.int32, (SH, SH), 0) % H)
         == (lax.broadcasted_iota(jnp.int32, (SH, SH), 1) % H)
         ).astype(jnp.float32)
    sub_nn = lax.broadcasted_iota(jnp.int32, (N, N), 0)
    lan_nn = lax.broadcasted_iota(jnp.int32, (N, N), 1)
    row_s = lax.broadcasted_iota(jnp.int32, (S, N), 0)
    lane_sh = lax.broadcasted_iota(jnp.int32, (N, SH), 1)
    svals = lane_sh // H

    C = D // 4
    T = (KC + 1) * C
    tile64 = (lax.broadcasted_iota(jnp.int32, (C, T), 1) % C
              == lax.broadcasted_iota(jnp.int32, (C, T), 0)
              ).astype(jnp.float32)
    prow = (lax.broadcasted_iota(jnp.int32, (T, C), 0) % C
            == lax.broadcasted_iota(jnp.int32, (T, C), 1)
            ).astype(jnp.float32)
    pcol = (lax.broadcasted_iota(jnp.int32, (H, SH), 1) % H
            == lax.broadcasted_iota(jnp.int32, (H, SH), 0)
            ).astype(jnp.float32)
    blk = (lax.broadcasted_iota(jnp.int32, (T, SH), 0) // C
           == lax.broadcasted_iota(jnp.int32, (T, SH), 1) // H
           ).astype(jnp.float32)
    w2b = _dot(_dot(prow, w2_ref[...], _DN), pcol) * blk
    rep448 = (lax.broadcasted_iota(jnp.int32, (S, T), 1) // C
              == lax.broadcasted_iota(jnp.int32, (S, T), 0)
              ).astype(jnp.float32)
    rb = rep448 * _dot(w1t_ref[1:2, :], tile64)
    b2tile = _dot(b2r_ref[...], pcol)
    a_flat = dist_flat * w1t_ref[0:1, :] + b1r_ref[...]
    a_tiled_flat = _dot(a_flat, tile64)
    lane_ns = lax.broadcasted_iota(jnp.int32, (N, S), 1)

    attn_scenes = []
    for sc in range(SPB):
        bg = pid * SPB + sc
        d_row = dist_full_ref[pl.ds(bg, 1), :]
        d_col = dist_flat[sc * N:(sc + 1) * N, :]
        tokens = tok_flat[sc * N:(sc + 1) * N, :]
        q = q_all[sc * N:(sc + 1) * N, :]

        lt = d_row < d_col
        eq = d_row == d_col
        cmp_t = (lt | (eq & (lan_nn < sub_nn))).astype(jnp.float32)
        rank_col = jnp.sum(cmp_t, axis=1, keepdims=True).astype(jnp.int32)
        lt2 = d_col < d_row
        cmp_r = (lt2 | (eq & (sub_nn < lan_nn))).astype(jnp.float32)
        rank_row = jnp.sum(cmp_r, axis=0, keepdims=True).astype(jnp.int32)

        r_col = jnp.minimum(rank_col, S - 1)
        onehot = ((row_s == rank_row) & (rank_row < KC + 1)
                  ).astype(jnp.float32)

        cand_tok = _dot(onehot, tokens)
        onehot_t = ((lane_ns == rank_col) & (lane_ns < KC + 1)
                    ).astype(jnp.float32)

        kt = _dot(kw_ref[...], cand_tok, _DN)
        v_cand = _dot(cand_tok, vw_ref[...], _DN)
        khead = _dot(kt, rep) * hm
        vheadt = _dot(rept, v_cand) * hmt

        sf = _dot(q, khead) * (1.0 / SCALE)

        d_cand_row = _dot(d_row, onehot_t)
        b_row = _dot(d_cand_row, rb)
        h1 = jnp.maximum(
            a_tiled_flat[sc * N:(sc + 1) * N, :] + b_row, 0.0)
        bias_all = _dot(h1, w2b) + b2tile

        rank_after_self = svals - (svals > r_col).astype(jnp.int32)
        valid = ((svals != r_col) & (rank_after_self < K_t)
                 & (svals < KC + 1))
        validf = valid.astype(jnp.float32)

        z = jnp.where(valid, sf + bias_all, -1e30)
        m = jnp.max(z, axis=1, keepdims=True)
        e = jnp.exp(z - m) * validf
        denom = _dot(e, g)
        w_all = e / denom

        attn = _dot(w_all, vheadt)
        out_ref[sc] = _dot(attn, ow_ref[...], _DN)
    del attn_scenes


@jax.jit
def kernel(tokens_B, ego_distances, ego_mask, ego_speed, q_w, k_w, v_w,
           ego_q_w, ego_k_w, ego_v_w, out_w, w1, b1, w2, b2):
    del ego_k_w, ego_v_w
    speed_row = ego_speed.reshape(1, B)
    dist_col = ego_distances.reshape(B, N, 1)
    mask_col = ego_mask.astype(jnp.float32).reshape(B, N, 1)
    qw2 = jnp.concatenate([q_w, ego_q_w], axis=0)
    w1t = w1.T
    b1r = b1.reshape(1, -1)
    b2r = b2.reshape(1, -1)

    const = lambda b: (0, 0)
    grid_spec = pl.GridSpec(
        grid=(B // SPB,),
        in_specs=[
            pl.BlockSpec((B, N), const),
            pl.BlockSpec((1, B), const),
            pl.BlockSpec((SPB, N, 1), lambda b: (b, 0, 0)),
            pl.BlockSpec((SPB, N, 1), lambda b: (b, 0, 0)),
            pl.BlockSpec((SPB, N, D), lambda b: (b, 0, 0)),
            pl.BlockSpec((2 * D, D), const),
            pl.BlockSpec((D, D), const),
            pl.BlockSpec((D, D), const),
            pl.BlockSpec((D, D), const),
            pl.BlockSpec((2, D // 4), const),
            pl.BlockSpec((1, D // 4), const),
            pl.BlockSpec((H, D // 4), const),
            pl.BlockSpec((1, H), const),
        ],
        out_specs=pl.BlockSpec((SPB, N, D), lambda b: (b, 0, 0)),
    )
    return pl.pallas_call(
        _fused_kernel,
        grid_spec=grid_spec,
        out_shape=jax.ShapeDtypeStruct((B, N, D), jnp.float32),
    )(ego_distances, speed_row, dist_col, mask_col, tokens_B,
      qw2, k_w, v_w, out_w, w1t, b1r, w2, b2r)
